# baseline (speedup 1.0000x reference)
import functools

import jax
import jax.numpy as jnp
from jax import lax
from jax.experimental import pallas as pl
from jax.experimental.pallas import tpu as pltpu
from jax.experimental.pallas import tpu_sc as plsc

D_MODEL = 1024
N_HEADS = 16
N_EXPERTS = 8
TOP_K = 2
D_FF = 4096
BATCH = 2
SEQ = 2048

N_TOK = BATCH * SEQ
N_ASSIGN = N_TOK * TOP_K
TILE = 256
PAD_ROWS = N_ASSIGN + N_EXPERTS * TILE
N_TILES = PAD_ROWS // TILE

NW = 32
TOK_PER_W = N_TOK // NW
SC_CHUNK = 64

_EPS = 1e-5
_NEG = jnp.finfo(jnp.float32).min


def _bf(a):
    return a.astype(jnp.bfloat16)


def _ln_qkv_body(x_ref, g_ref, b_ref, wq_ref, wk_ref, wv_ref, q_ref, k_ref, v_ref):
    xb = x_ref[...]
    mu = jnp.mean(xb, axis=1, keepdims=True)
    var = jnp.mean((xb - mu) ** 2, axis=1, keepdims=True)
    h = (xb - mu) / jnp.sqrt(var + _EPS) * g_ref[...] + b_ref[...]
    hb = _bf(h)
    q_ref[...] = jnp.dot(hb, wq_ref[...], preferred_element_type=jnp.float32)
    k_ref[...] = jnp.dot(hb, wk_ref[...], preferred_element_type=jnp.float32)
    v_ref[...] = jnp.dot(hb, wv_ref[...], preferred_element_type=jnp.float32)


def _ln_qkv(x2d, g, b, Wq, Wk, Wv):
    nt = N_TOK // TILE
    row = pl.BlockSpec((TILE, D_MODEL), lambda i: (i, 0))
    full = pl.BlockSpec((D_MODEL, D_MODEL), lambda i: (0, 0))
    vec = pl.BlockSpec((1, D_MODEL), lambda i: (0, 0))
    return pl.pallas_call(
        _ln_qkv_body,
        grid=(nt,),
        in_specs=[row, vec, vec, full, full, full],
        out_specs=[row, row, row],
        out_shape=[jax.ShapeDtypeStruct((N_TOK, D_MODEL), jnp.float32)] * 3,
    )(x2d, g.reshape(1, -1), b.reshape(1, -1), _bf(Wq), _bf(Wk), _bf(Wv))


def _attn_body(q_ref, k_ref, v_ref, o_ref):
    qi = pl.program_id(1)
    QT = SEQ // 8
    q = _bf(q_ref[0])
    rows = lax.broadcasted_iota(jnp.int32, (QT, QT), 0)
    cols = lax.broadcasted_iota(jnp.int32, (QT, QT), 1)
    upper = cols > rows

    def step(kc, carry):
        m, l, acc = carry
        kch = _bf(k_ref[0, pl.ds(kc * QT, QT), :])
        vch = _bf(v_ref[0, pl.ds(kc * QT, QT), :])
        s = lax.dot_general(q, kch, (((1,), (1,)), ((), ())),
                            preferred_element_type=jnp.float32) * (1.0 / 8.0)
        s = jnp.where(jnp.logical_and(kc == qi, upper), _NEG, s)
        mc = jnp.max(s, axis=1, keepdims=True)
        mn = jnp.maximum(m, mc)
        p = jnp.exp(s - mn)
        corr = jnp.exp(m - mn)
        l = l * corr + jnp.sum(p, axis=1, keepdims=True)
        acc = acc * corr + jnp.dot(_bf(p), vch,
                                   preferred_element_type=jnp.float32)
        return mn, l, acc

    m0 = jnp.full((QT, 1), _NEG, jnp.float32)
    l0 = jnp.zeros((QT, 1), jnp.float32)
    a0 = jnp.zeros((QT, D_MODEL // N_HEADS), jnp.float32)
    m, l, acc = lax.fori_loop(0, qi + 1, step, (m0, l0, a0))
    o_ref[0] = acc / l


def _attention(qt, kt, vt):
    BH = BATCH * N_HEADS
    DH = D_MODEL // N_HEADS
    QT = SEQ // 8
    qspec = pl.BlockSpec((1, QT, DH), lambda b, i: (b, i, 0))
    kvspec = pl.BlockSpec((1, SEQ, DH), lambda b, i: (b, 0, 0))
    return pl.pallas_call(
        _attn_body,
        grid=(BH, 8),
        in_specs=[qspec, kvspec, kvspec],
        out_specs=qspec,
        out_shape=jax.ShapeDtypeStruct((BH, SEQ, DH), jnp.float32),
    )(qt, kt, vt)


def _post_body(x_ref, c_ref, wo_ref, g_ref, b_ref, wg_ref,
               xr_ref, z_ref, i0_ref, i1_ref, w0_ref, w1_ref):
    xr = x_ref[...] + jnp.dot(_bf(c_ref[...]), wo_ref[...],
                              preferred_element_type=jnp.float32)
    xr_ref[...] = xr
    mu = jnp.mean(xr, axis=1, keepdims=True)
    var = jnp.mean((xr - mu) ** 2, axis=1, keepdims=True)
    z = (xr - mu) / jnp.sqrt(var + _EPS) * g_ref[...] + b_ref[...]
    z_ref[...] = z
    logits = jnp.dot(_bf(z), wg_ref[...], preferred_element_type=jnp.float32)
    ei = lax.broadcasted_iota(jnp.int32, logits.shape, 1)
    m0 = jnp.max(logits, axis=1, keepdims=True)
    i0 = jnp.min(jnp.where(logits == m0, ei, N_EXPERTS), axis=1, keepdims=True)
    masked = jnp.where(ei == i0, _NEG, logits)
    m1 = jnp.max(masked, axis=1, keepdims=True)
    i1 = jnp.min(jnp.where(masked == m1, ei, N_EXPERTS), axis=1, keepdims=True)
    w0 = 1.0 / (1.0 + jnp.exp(m1 - m0))
    i0_ref[...] = i0
    i1_ref[...] = i1
    w0_ref[...] = w0
    w1_ref[...] = 1.0 - w0


def _post_attn(x2d, ctx2d, Wo, g, b, Wg):
    nt = N_TOK // TILE
    row = pl.BlockSpec((TILE, D_MODEL), lambda i: (i, 0))
    full = pl.BlockSpec((D_MODEL, D_MODEL), lambda i: (0, 0))
    vec = pl.BlockSpec((1, D_MODEL), lambda i: (0, 0))
    gspec = pl.BlockSpec((D_MODEL, N_EXPERTS), lambda i: (0, 0))
    col_f = pl.BlockSpec((TILE, 1), lambda i: (i, 0))
    return pl.pallas_call(
        _post_body,
        grid=(nt,),
        in_specs=[row, row, full, vec, vec, gspec],
        out_specs=[row, row, col_f, col_f, col_f, col_f],
        out_shape=[
            jax.ShapeDtypeStruct((N_TOK, D_MODEL), jnp.float32),
            jax.ShapeDtypeStruct((N_TOK, D_MODEL), jnp.float32),
            jax.ShapeDtypeStruct((N_TOK, 1), jnp.int32),
            jax.ShapeDtypeStruct((N_TOK, 1), jnp.int32),
            jax.ShapeDtypeStruct((N_TOK, 1), jnp.float32),
            jax.ShapeDtypeStruct((N_TOK, 1), jnp.float32),
        ],
    )(x2d, ctx2d, _bf(Wo), g.reshape(1, -1), b.reshape(1, -1), _bf(Wg))


def _route_body(e_ref, p_ref, te_ref):
    arr = e_ref[...]
    r128 = lax.broadcasted_iota(jnp.int32, (128, 128), 0)
    c128 = lax.broadcasted_iota(jnp.int32, (128, 128), 1)
    su = (r128 < c128).astype(jnp.float32)
    r64 = lax.broadcasted_iota(jnp.int32, (64, 64), 0)
    c64 = lax.broadcasted_iota(jnp.int32, (64, 64), 1)
    sl = (c64 < r64).astype(jnp.float32)
    ones = jnp.ones((128, 128), jnp.float32)
    p_acc = jnp.zeros((64, 128), jnp.float32)
    off = jnp.int32(0)
    offs = []
    for e in range(N_EXPERTS):
        oh = (arr == e).astype(jnp.float32)
        inrow = jnp.dot(oh, su, preferred_element_type=jnp.float32,
                        precision=lax.Precision.HIGHEST)
        rowpre = jnp.dot(jnp.dot(sl, oh, preferred_element_type=jnp.float32,
                                 precision=lax.Precision.HIGHEST),
                         ones, preferred_element_type=jnp.float32,
                         precision=lax.Precision.HIGHEST)
        rank = inrow + rowpre
        p_acc = p_acc + oh * (rank + off.astype(jnp.float32))
        offs.append(off)
        cnt = jnp.sum(oh).astype(jnp.int32)
        off = off + (((cnt + TILE - 1) >> 8) << 8)
    p_ref[...] = p_acc.astype(jnp.int32)
    ti = lax.broadcasted_iota(jnp.int32, (8, 128), 1) * TILE
    te = jnp.zeros((8, 128), jnp.int32) - 1
    for e in range(N_EXPERTS):
        te = te + (ti >= offs[e]).astype(jnp.int32)
    te_ref[...] = jnp.where(ti < off, te, -1)


def _routing(e_all):
    return pl.pallas_call(
        _route_body,
        out_shape=[jax.ShapeDtypeStruct((64, 128), jnp.int32),
                   jax.ShapeDtypeStruct((8, 128), jnp.int32)],
    )(e_all)


def _dispatch_sc(z, p0, p1):
    mesh = plsc.VectorSubcoreMesh(core_axis_name="c", subcore_axis_name="s",
                                  num_cores=2, num_subcores=16)

    @functools.partial(
        pl.kernel, mesh=mesh,
        out_type=jax.ShapeDtypeStruct((PAD_ROWS, D_MODEL), jnp.float32),
        scratch_types=[
            pltpu.VMEM((SC_CHUNK,), jnp.int32),
            pltpu.VMEM((SC_CHUNK, D_MODEL), jnp.float32),
            pltpu.SemaphoreType.DMA,
        ],
    )
    def k(z_hbm, p0_hbm, p1_hbm, xs_hbm, idx_v, rows_v, sem):
        wid = lax.axis_index("s") * 2 + lax.axis_index("c")
        base = wid * TOK_PER_W
        for p_hbm in (p0_hbm, p1_hbm):
            for ci in range(TOK_PER_W // SC_CHUNK):
                rb = base + ci * SC_CHUNK
                pltpu.sync_copy(z_hbm.at[pl.ds(rb, SC_CHUNK)], rows_v)
                pltpu.sync_copy(p_hbm.at[wid, pl.ds(ci * SC_CHUNK, SC_CHUNK)],
                                idx_v)
                pltpu.async_copy(rows_v, xs_hbm.at[idx_v], sem).wait()

    return k(z, p0, p1)


def _gather_sc(y, p0, p1):
    mesh = plsc.VectorSubcoreMesh(core_axis_name="c", subcore_axis_name="s",
                                  num_cores=2, num_subcores=16)

    @functools.partial(
        pl.kernel, mesh=mesh,
        out_type=[jax.ShapeDtypeStruct((N_TOK, D_MODEL), jnp.float32),
                  jax.ShapeDtypeStruct((N_TOK, D_MODEL), jnp.float32)],
        scratch_types=[
            pltpu.VMEM((SC_CHUNK,), jnp.int32),
            pltpu.VMEM((SC_CHUNK, D_MODEL), jnp.float32),
            pltpu.SemaphoreType.DMA,
        ],
    )
    def k(y_hbm, p0_hbm, p1_hbm, c0_hbm, c1_hbm, idx_v, rows_v, sem):
        wid = lax.axis_index("s") * 2 + lax.axis_index("c")
        base = wid * TOK_PER_W
        for p_hbm, c_hbm in ((p0_hbm, c0_hbm), (p1_hbm, c1_hbm)):
            for ci in range(TOK_PER_W // SC_CHUNK):
                rb = base + ci * SC_CHUNK
                pltpu.sync_copy(p_hbm.at[wid, pl.ds(ci * SC_CHUNK, SC_CHUNK)],
                                idx_v)
                pltpu.async_copy(y_hbm.at[idx_v], rows_v, sem).wait()
                pltpu.sync_copy(rows_v, c_hbm.at[pl.ds(rb, SC_CHUNK)])

    return k(y, p0, p1)


def _ffn_body(te_ref, xs_ref, w1_ref, b1_ref, w2_ref, b2_ref, y_ref):
    @pl.when(te_ref[pl.program_id(0)] >= 0)
    def _():
        xb = xs_ref[...].astype(jnp.bfloat16)
        h = (jnp.dot(xb, w1_ref[0], preferred_element_type=jnp.float32)
             + b1_ref[0])
        g = jax.nn.gelu(h).astype(jnp.bfloat16)
        y_ref[...] = (jnp.dot(g, w2_ref[0], preferred_element_type=jnp.float32)
                      + b2_ref[0])


def _ffn_grouped(xs, te, W1b, b1, W2b, b2):
    grid_spec = pltpu.PrefetchScalarGridSpec(
        num_scalar_prefetch=1,
        grid=(N_TILES,),
        in_specs=[
            pl.BlockSpec((TILE, D_MODEL), lambda i, te_ref: (i, 0)),
            pl.BlockSpec((1, D_MODEL, D_FF),
                         lambda i, te_ref: (jnp.maximum(te_ref[i], 0), 0, 0)),
            pl.BlockSpec((1, 1, D_FF),
                         lambda i, te_ref: (jnp.maximum(te_ref[i], 0), 0, 0)),
            pl.BlockSpec((1, D_FF, D_MODEL),
                         lambda i, te_ref: (jnp.maximum(te_ref[i], 0), 0, 0)),
            pl.BlockSpec((1, 1, D_MODEL),
                         lambda i, te_ref: (jnp.maximum(te_ref[i], 0), 0, 0)),
        ],
        out_specs=pl.BlockSpec((TILE, D_MODEL), lambda i, te_ref: (i, 0)),
    )
    return pl.pallas_call(
        _ffn_body,
        grid_spec=grid_spec,
        out_shape=jax.ShapeDtypeStruct((PAD_ROWS, D_MODEL), jnp.float32),
    )(te, xs, W1b, b1.reshape(N_EXPERTS, 1, D_FF), W2b,
      b2.reshape(N_EXPERTS, 1, D_MODEL))


def _combine_body(xr_ref, c0_ref, c1_ref, w0_ref, w1_ref, o_ref):
    o_ref[...] = (xr_ref[...] + w0_ref[...] * c0_ref[...]
                  + w1_ref[...] * c1_ref[...])


def _combine(xr, c0, c1, w0, w1):
    nt = N_TOK // TILE
    row = pl.BlockSpec((TILE, D_MODEL), lambda i: (i, 0))
    col = pl.BlockSpec((TILE, 1), lambda i: (i, 0))
    return pl.pallas_call(
        _combine_body,
        grid=(nt,),
        in_specs=[row, row, row, col, col],
        out_specs=row,
        out_shape=jax.ShapeDtypeStruct((N_TOK, D_MODEL), jnp.float32),
    )(xr, c0, c1, w0, w1)


def kernel(x, ln1_g, ln1_b, ln2_g, ln2_b, Wq, Wk, Wv, Wo, Wg, W1, b1, W2, b2):
    B, S, d = x.shape
    H = N_HEADS
    dh = d // H
    x2d = x.reshape(B * S, d)

    q, k, v = _ln_qkv(x2d, ln1_g, ln1_b, Wq, Wk, Wv)
    qt = q.reshape(B, S, H, dh).transpose(0, 2, 1, 3).reshape(B * H, S, dh)
    kt = k.reshape(B, S, H, dh).transpose(0, 2, 1, 3).reshape(B * H, S, dh)
    vt = v.reshape(B, S, H, dh).transpose(0, 2, 1, 3).reshape(B * H, S, dh)
    ctx = _attention(qt, kt, vt)
    ctx2d = ctx.reshape(B, H, S, dh).transpose(0, 2, 1, 3).reshape(B * S, d)

    xr, z, i0, i1, w0, w1 = _post_attn(x2d, ctx2d, Wo, ln2_g, ln2_b, Wg)

    e_all = jnp.concatenate(
        [i0.reshape(32, 128), i1.reshape(32, 128)], axis=0)
    p, te = _routing(e_all)
    p0 = p[:32]
    p1 = p[32:]
    te_flat = te[0]

    xs = _dispatch_sc(z, p0, p1)
    y = _ffn_grouped(xs, te_flat, W1.astype(jnp.bfloat16), b1,
                     W2.astype(jnp.bfloat16), b2)
    c0, c1 = _gather_sc(y, p0, p1)
    out = _combine(xr, c0, c1, w0, w1)
    return out.reshape(B, S, d)


# speedup vs baseline: 1.2035x; 1.2035x over previous
import functools

import jax
import jax.numpy as jnp
from jax import lax
from jax.experimental import pallas as pl
from jax.experimental.pallas import tpu as pltpu
from jax.experimental.pallas import tpu_sc as plsc

D_MODEL = 1024
N_HEADS = 16
N_EXPERTS = 8
TOP_K = 2
D_FF = 4096
BATCH = 2
SEQ = 2048

N_TOK = BATCH * SEQ
N_ASSIGN = N_TOK * TOP_K
TILE = 256
PAD_ROWS = N_ASSIGN + N_EXPERTS * TILE
N_TILES = PAD_ROWS // TILE

NW = 32
TOK_PER_W = N_TOK // NW
SC_CHUNK = 64

_EPS = 1e-5
_NEG = jnp.finfo(jnp.float32).min


def _bf(a):
    return a.astype(jnp.bfloat16)


def _ln_qkv_body(x_ref, g_ref, b_ref, wq_ref, wk_ref, wv_ref, q_ref, k_ref, v_ref):
    xb = x_ref[...]
    mu = jnp.mean(xb, axis=1, keepdims=True)
    var = jnp.mean((xb - mu) ** 2, axis=1, keepdims=True)
    h = (xb - mu) / jnp.sqrt(var + _EPS) * g_ref[...] + b_ref[...]
    hb = _bf(h)
    q_ref[...] = jnp.dot(hb, wq_ref[...], preferred_element_type=jnp.float32)
    k_ref[...] = jnp.dot(hb, wk_ref[...], preferred_element_type=jnp.float32)
    v_ref[...] = jnp.dot(hb, wv_ref[...], preferred_element_type=jnp.float32)


def _ln_qkv(x2d, g, b, Wq, Wk, Wv):
    nt = N_TOK // TILE
    row = pl.BlockSpec((TILE, D_MODEL), lambda i: (i, 0))
    full = pl.BlockSpec((D_MODEL, D_MODEL), lambda i: (0, 0))
    vec = pl.BlockSpec((1, D_MODEL), lambda i: (0, 0))
    return pl.pallas_call(
        _ln_qkv_body,
        grid=(nt,),
        in_specs=[row, vec, vec, full, full, full],
        out_specs=[row, row, row],
        out_shape=[jax.ShapeDtypeStruct((N_TOK, D_MODEL), jnp.float32)] * 3,
    )(x2d, g.reshape(1, -1), b.reshape(1, -1), _bf(Wq), _bf(Wk), _bf(Wv))


def _attn_body(q_ref, k_ref, v_ref, o_ref):
    qi = pl.program_id(1)
    q = _bf(q_ref[0])
    k = _bf(k_ref[0])
    v = _bf(v_ref[0])
    s = lax.dot_general(q, k, (((1,), (1,)), ((), ())),
                        preferred_element_type=jnp.float32) * (1.0 / 8.0)
    rows = lax.broadcasted_iota(jnp.int32, s.shape, 0) + qi * (SEQ // 8)
    cols = lax.broadcasted_iota(jnp.int32, s.shape, 1)
    s = jnp.where(cols <= rows, s, _NEG)
    m = jnp.max(s, axis=1, keepdims=True)
    e = jnp.exp(s - m)
    p = e / jnp.sum(e, axis=1, keepdims=True)
    o_ref[0] = jnp.dot(_bf(p), v, preferred_element_type=jnp.float32)


def _attention(qt, kt, vt):
    BH = BATCH * N_HEADS
    DH = D_MODEL // N_HEADS
    QT = SEQ // 8
    qspec = pl.BlockSpec((1, QT, DH), lambda b, i: (b, i, 0))
    kvspec = pl.BlockSpec((1, SEQ, DH), lambda b, i: (b, 0, 0))
    return pl.pallas_call(
        _attn_body,
        grid=(BH, 8),
        in_specs=[qspec, kvspec, kvspec],
        out_specs=qspec,
        out_shape=jax.ShapeDtypeStruct((BH, SEQ, DH), jnp.float32),
    )(qt, kt, vt)


def _post_body(x_ref, c_ref, wo_ref, g_ref, b_ref, wg_ref,
               xr_ref, z_ref, i0_ref, i1_ref, w0_ref, w1_ref):
    xr = x_ref[...] + jnp.dot(_bf(c_ref[...]), wo_ref[...],
                              preferred_element_type=jnp.float32)
    xr_ref[...] = xr
    mu = jnp.mean(xr, axis=1, keepdims=True)
    var = jnp.mean((xr - mu) ** 2, axis=1, keepdims=True)
    z = (xr - mu) / jnp.sqrt(var + _EPS) * g_ref[...] + b_ref[...]
    z_ref[...] = z
    logits = jnp.dot(_bf(z), wg_ref[...], preferred_element_type=jnp.float32)
    ei = lax.broadcasted_iota(jnp.int32, logits.shape, 1)
    m0 = jnp.max(logits, axis=1, keepdims=True)
    i0 = jnp.min(jnp.where(logits == m0, ei, N_EXPERTS), axis=1, keepdims=True)
    masked = jnp.where(ei == i0, _NEG, logits)
    m1 = jnp.max(masked, axis=1, keepdims=True)
    i1 = jnp.min(jnp.where(masked == m1, ei, N_EXPERTS), axis=1, keepdims=True)
    w0 = 1.0 / (1.0 + jnp.exp(m1 - m0))
    i0_ref[...] = i0
    i1_ref[...] = i1
    w0_ref[...] = w0
    w1_ref[...] = 1.0 - w0


def _post_attn(x2d, ctx2d, Wo, g, b, Wg):
    nt = N_TOK // TILE
    row = pl.BlockSpec((TILE, D_MODEL), lambda i: (i, 0))
    full = pl.BlockSpec((D_MODEL, D_MODEL), lambda i: (0, 0))
    vec = pl.BlockSpec((1, D_MODEL), lambda i: (0, 0))
    gspec = pl.BlockSpec((D_MODEL, N_EXPERTS), lambda i: (0, 0))
    col_f = pl.BlockSpec((TILE, 1), lambda i: (i, 0))
    return pl.pallas_call(
        _post_body,
        grid=(nt,),
        in_specs=[row, row, full, vec, vec, gspec],
        out_specs=[row, row, col_f, col_f, col_f, col_f],
        out_shape=[
            jax.ShapeDtypeStruct((N_TOK, D_MODEL), jnp.float32),
            jax.ShapeDtypeStruct((N_TOK, D_MODEL), jnp.float32),
            jax.ShapeDtypeStruct((N_TOK, 1), jnp.int32),
            jax.ShapeDtypeStruct((N_TOK, 1), jnp.int32),
            jax.ShapeDtypeStruct((N_TOK, 1), jnp.float32),
            jax.ShapeDtypeStruct((N_TOK, 1), jnp.float32),
        ],
    )(x2d, ctx2d, _bf(Wo), g.reshape(1, -1), b.reshape(1, -1), _bf(Wg))


def _route_body(e_ref, p_ref, te_ref):
    arr = e_ref[...]
    r128 = lax.broadcasted_iota(jnp.int32, (128, 128), 0)
    c128 = lax.broadcasted_iota(jnp.int32, (128, 128), 1)
    su = (r128 < c128).astype(jnp.float32)
    r64 = lax.broadcasted_iota(jnp.int32, (64, 64), 0)
    c64 = lax.broadcasted_iota(jnp.int32, (64, 64), 1)
    sl = (c64 < r64).astype(jnp.float32)
    ones = jnp.ones((128, 128), jnp.float32)
    p_acc = jnp.zeros((64, 128), jnp.float32)
    off = jnp.int32(0)
    offs = []
    for e in range(N_EXPERTS):
        oh = (arr == e).astype(jnp.float32)
        inrow = jnp.dot(oh, su, preferred_element_type=jnp.float32,
                        precision=lax.Precision.HIGHEST)
        rowpre = jnp.dot(jnp.dot(sl, oh, preferred_element_type=jnp.float32,
                                 precision=lax.Precision.HIGHEST),
                         ones, preferred_element_type=jnp.float32,
                         precision=lax.Precision.HIGHEST)
        rank = inrow + rowpre
        p_acc = p_acc + oh * (rank + off.astype(jnp.float32))
        offs.append(off)
        cnt = jnp.sum(oh).astype(jnp.int32)
        off = off + (((cnt + TILE - 1) >> 8) << 8)
    p_ref[...] = p_acc.astype(jnp.int32)
    ti = lax.broadcasted_iota(jnp.int32, (8, 128), 1) * TILE
    te = jnp.zeros((8, 128), jnp.int32) - 1
    for e in range(N_EXPERTS):
        te = te + (ti >= offs[e]).astype(jnp.int32)
    te_ref[...] = jnp.where(ti < off, te, -1)


def _routing(e_all):
    return pl.pallas_call(
        _route_body,
        out_shape=[jax.ShapeDtypeStruct((64, 128), jnp.int32),
                   jax.ShapeDtypeStruct((8, 128), jnp.int32)],
    )(e_all)


def _dispatch_sc(z, p0, p1):
    mesh = plsc.VectorSubcoreMesh(core_axis_name="c", subcore_axis_name="s",
                                  num_cores=2, num_subcores=16)

    @functools.partial(
        pl.kernel, mesh=mesh,
        out_type=jax.ShapeDtypeStruct((PAD_ROWS, D_MODEL), jnp.float32),
        scratch_types=[
            pltpu.VMEM((SC_CHUNK,), jnp.int32),
            pltpu.VMEM((SC_CHUNK, D_MODEL), jnp.float32),
            pltpu.SemaphoreType.DMA,
        ],
    )
    def k(z_hbm, p0_hbm, p1_hbm, xs_hbm, idx_v, rows_v, sem):
        wid = lax.axis_index("s") * 2 + lax.axis_index("c")
        base = wid * TOK_PER_W
        for p_hbm in (p0_hbm, p1_hbm):
            for ci in range(TOK_PER_W // SC_CHUNK):
                rb = base + ci * SC_CHUNK
                pltpu.sync_copy(z_hbm.at[pl.ds(rb, SC_CHUNK)], rows_v)
                pltpu.sync_copy(p_hbm.at[wid, pl.ds(ci * SC_CHUNK, SC_CHUNK)],
                                idx_v)
                pltpu.async_copy(rows_v, xs_hbm.at[idx_v], sem).wait()

    return k(z, p0, p1)


def _gather_sc(y, p0, p1):
    mesh = plsc.VectorSubcoreMesh(core_axis_name="c", subcore_axis_name="s",
                                  num_cores=2, num_subcores=16)

    @functools.partial(
        pl.kernel, mesh=mesh,
        out_type=[jax.ShapeDtypeStruct((N_TOK, D_MODEL), jnp.float32),
                  jax.ShapeDtypeStruct((N_TOK, D_MODEL), jnp.float32)],
        scratch_types=[
            pltpu.VMEM((SC_CHUNK,), jnp.int32),
            pltpu.VMEM((SC_CHUNK, D_MODEL), jnp.float32),
            pltpu.SemaphoreType.DMA,
        ],
    )
    def k(y_hbm, p0_hbm, p1_hbm, c0_hbm, c1_hbm, idx_v, rows_v, sem):
        wid = lax.axis_index("s") * 2 + lax.axis_index("c")
        base = wid * TOK_PER_W
        for p_hbm, c_hbm in ((p0_hbm, c0_hbm), (p1_hbm, c1_hbm)):
            for ci in range(TOK_PER_W // SC_CHUNK):
                rb = base + ci * SC_CHUNK
                pltpu.sync_copy(p_hbm.at[wid, pl.ds(ci * SC_CHUNK, SC_CHUNK)],
                                idx_v)
                pltpu.async_copy(y_hbm.at[idx_v], rows_v, sem).wait()
                pltpu.sync_copy(rows_v, c_hbm.at[pl.ds(rb, SC_CHUNK)])

    return k(y, p0, p1)


def _ffn_body(te_ref, xs_ref, w1_ref, b1_ref, w2_ref, b2_ref, y_ref):
    @pl.when(te_ref[pl.program_id(0)] >= 0)
    def _():
        xb = xs_ref[...].astype(jnp.bfloat16)
        h = (jnp.dot(xb, w1_ref[0], preferred_element_type=jnp.float32)
             + b1_ref[0])
        g = jax.nn.gelu(h).astype(jnp.bfloat16)
        y_ref[...] = (jnp.dot(g, w2_ref[0], preferred_element_type=jnp.float32)
                      + b2_ref[0])


def _ffn_grouped(xs, te, W1b, b1, W2b, b2):
    grid_spec = pltpu.PrefetchScalarGridSpec(
        num_scalar_prefetch=1,
        grid=(N_TILES,),
        in_specs=[
            pl.BlockSpec((TILE, D_MODEL), lambda i, te_ref: (i, 0)),
            pl.BlockSpec((1, D_MODEL, D_FF),
                         lambda i, te_ref: (jnp.maximum(te_ref[i], 0), 0, 0)),
            pl.BlockSpec((1, 1, D_FF),
                         lambda i, te_ref: (jnp.maximum(te_ref[i], 0), 0, 0)),
            pl.BlockSpec((1, D_FF, D_MODEL),
                         lambda i, te_ref: (jnp.maximum(te_ref[i], 0), 0, 0)),
            pl.BlockSpec((1, 1, D_MODEL),
                         lambda i, te_ref: (jnp.maximum(te_ref[i], 0), 0, 0)),
        ],
        out_specs=pl.BlockSpec((TILE, D_MODEL), lambda i, te_ref: (i, 0)),
    )
    return pl.pallas_call(
        _ffn_body,
        grid_spec=grid_spec,
        out_shape=jax.ShapeDtypeStruct((PAD_ROWS, D_MODEL), jnp.float32),
    )(te, xs, W1b, b1.reshape(N_EXPERTS, 1, D_FF), W2b,
      b2.reshape(N_EXPERTS, 1, D_MODEL))


def _combine_body(xr_ref, c0_ref, c1_ref, w0_ref, w1_ref, o_ref):
    o_ref[...] = (xr_ref[...] + w0_ref[...] * c0_ref[...]
                  + w1_ref[...] * c1_ref[...])


def _combine(xr, c0, c1, w0, w1):
    nt = N_TOK // TILE
    row = pl.BlockSpec((TILE, D_MODEL), lambda i: (i, 0))
    col = pl.BlockSpec((TILE, 1), lambda i: (i, 0))
    return pl.pallas_call(
        _combine_body,
        grid=(nt,),
        in_specs=[row, row, row, col, col],
        out_specs=row,
        out_shape=jax.ShapeDtypeStruct((N_TOK, D_MODEL), jnp.float32),
    )(xr, c0, c1, w0, w1)


def kernel(x, ln1_g, ln1_b, ln2_g, ln2_b, Wq, Wk, Wv, Wo, Wg, W1, b1, W2, b2):
    B, S, d = x.shape
    H = N_HEADS
    dh = d // H
    x2d = x.reshape(B * S, d)

    q, k, v = _ln_qkv(x2d, ln1_g, ln1_b, Wq, Wk, Wv)
    qt = q.reshape(B, S, H, dh).transpose(0, 2, 1, 3).reshape(B * H, S, dh)
    kt = k.reshape(B, S, H, dh).transpose(0, 2, 1, 3).reshape(B * H, S, dh)
    vt = v.reshape(B, S, H, dh).transpose(0, 2, 1, 3).reshape(B * H, S, dh)
    ctx = _attention(qt, kt, vt)
    ctx2d = ctx.reshape(B, H, S, dh).transpose(0, 2, 1, 3).reshape(B * S, d)

    xr, z, i0, i1, w0, w1 = _post_attn(x2d, ctx2d, Wo, ln2_g, ln2_b, Wg)

    e_all = jnp.concatenate(
        [i0.reshape(32, 128), i1.reshape(32, 128)], axis=0)
    p, te = _routing(e_all)
    p0 = p[:32]
    p1 = p[32:]
    te_flat = te[0]

    xs = _dispatch_sc(z, p0, p1)
    y = _ffn_grouped(xs, te_flat, W1.astype(jnp.bfloat16), b1,
                     W2.astype(jnp.bfloat16), b2)
    c0, c1 = _gather_sc(y, p0, p1)
    out = _combine(xr, c0, c1, w0, w1)
    return out.reshape(B, S, d)
